# baseline (speedup 1.0000x reference)
import functools

import jax
import jax.numpy as jnp
import numpy as np
from jax.experimental import pallas as pl
from jax.experimental.pallas import tpu as pltpu

_SCALE = 128.0
_CLIP = 100.0
_GRP = 8
_CIN = 16
_COUT = 17
_LIN = _GRP * _CIN
_LOUT = _GRP * _COUT


def _xtype(c: int) -> bool:
    return (c % 2 == 0) == (c >= 4)


def _w16() -> np.ndarray:
    w = np.zeros((16, 16), dtype=np.float32)
    w[1, 0], w[3, 0] = 1.0, -0.5
    w[0, 1], w[2, 1] = 1.0, -0.5
    w[1, 2], w[3, 2] = 1.0, 0.5
    w[0, 3], w[2, 3] = 1.0, 0.5
    for c in range(4, 16):
        w[c, c] = 1.0
    return w / _SCALE


def _consts():
    w16 = _w16()
    w = np.zeros((_LIN, _LOUT), dtype=np.float32)
    for a in range(_GRP):
        for c in range(_CIN):
            for j in range(_CIN):
                w[_CIN * a + j, _COUT * a + c] = w16[j, c]
    es = np.zeros((4 * _GRP, _LOUT), dtype=np.float32)
    eo = np.zeros((4 * _GRP, _LOUT), dtype=np.float32)
    for a in range(_GRP):
        for c in range(_CIN):
            es[4 * a + (2 if _xtype(c) else 3), _COUT * a + c] = 1.0
            eo[4 * a + (0 if _xtype(c) else 1), _COUT * a + c] = 1.0
    q = np.zeros((_GRP, _LOUT), dtype=np.float32)
    for a in range(_GRP):
        q[a, _COUT * a + _CIN] = 1.0
    return w, es, eo, q


def _body(raw_ref, score_ref, anc_ref, w_ref, es_ref, eo_ref, q_ref, out_ref,
          sp_ref, op_ref):
    @pl.when(pl.program_id(0) == 0)
    def _init():
        anc = anc_ref[...]
        sp_ref[...] = jax.lax.dot_general(
            anc, es_ref[...], (((1,), (0,)), ((), ())),
            precision=jax.lax.Precision.HIGHEST,
            preferred_element_type=jnp.float32)
        op_ref[...] = jax.lax.dot_general(
            anc, eo_ref[...], (((1,), (0,)), ((), ())),
            precision=jax.lax.Precision.HIGHEST,
            preferred_element_type=jnp.float32)

    xp = jax.lax.dot_general(
        raw_ref[0].astype(jnp.bfloat16), w_ref[...], (((1,), (0,)), ((), ())),
        preferred_element_type=jnp.float32)
    sig = jax.nn.sigmoid(jnp.clip(score_ref[0], -_CLIP, _CLIP))
    sq = jax.lax.dot_general(
        sig.astype(jnp.bfloat16), q_ref[...], (((1,), (0,)), ((), ())),
        preferred_element_type=jnp.float32)
    out_ref[0] = xp * sp_ref[...] + (op_ref[...] + sq)


@jax.jit
def _run(raw_box_tensor, raw_score_tensor, anchors):
    b, a, cin = raw_box_tensor.shape
    rows = a // _GRP
    rawv = raw_box_tensor.reshape(b, rows, _LIN)
    scorev = raw_score_tensor.reshape(b, rows, _GRP)
    ancv = anchors.reshape(rows, 4 * _GRP)
    w, es, eo, q = _consts()
    out = pl.pallas_call(
        _body,
        grid=(b,),
        in_specs=[
            pl.BlockSpec((1, rows, _LIN), lambda i: (i, 0, 0)),
            pl.BlockSpec((1, rows, _GRP), lambda i: (i, 0, 0)),
            pl.BlockSpec((rows, 4 * _GRP), lambda i: (0, 0)),
            pl.BlockSpec((_LIN, _LOUT), lambda i: (0, 0)),
            pl.BlockSpec((4 * _GRP, _LOUT), lambda i: (0, 0)),
            pl.BlockSpec((4 * _GRP, _LOUT), lambda i: (0, 0)),
            pl.BlockSpec((_GRP, _LOUT), lambda i: (0, 0)),
        ],
        out_specs=pl.BlockSpec((1, rows, _LOUT), lambda i: (i, 0, 0)),
        out_shape=jax.ShapeDtypeStruct((b, rows, _LOUT), jnp.float32),
        scratch_shapes=[
            pltpu.VMEM((rows, _LOUT), jnp.float32),
            pltpu.VMEM((rows, _LOUT), jnp.float32),
        ],
    )(rawv, scorev, ancv, jnp.asarray(w, dtype=jnp.bfloat16), jnp.asarray(es),
      jnp.asarray(eo), jnp.asarray(q, dtype=jnp.bfloat16))
    return out.reshape(b, a, _COUT)


def kernel(raw_box_tensor, raw_score_tensor, anchors):
    return _run(raw_box_tensor, raw_score_tensor, anchors)


# speedup vs baseline: 2.2511x; 2.2511x over previous
import functools

import jax
import jax.numpy as jnp
from jax import lax
from jax.experimental import pallas as pl
from jax.experimental.pallas import tpu as pltpu
from jax.experimental.pallas import tpu_sc as plsc

_CLIP = 100.0
_INV = 1.0 / 128.0
_CH = 80


def _sc_run(raw, scr, ancp):
    b, a, cin = raw.shape
    info = plsc.get_sparse_core_info()
    nw = info.num_cores * info.num_subcores
    halves = nw // b
    per_w = a // halves
    n_chunks = per_w // _CH
    mesh = plsc.VectorSubcoreMesh(core_axis_name="c", subcore_axis_name="s")

    @functools.partial(
        pl.kernel,
        mesh=mesh,
        out_type=jax.ShapeDtypeStruct((b, a, cin + 1), jnp.float32),
        scratch_types=[
            pltpu.VMEM((1, _CH, 16), jnp.float32),
            pltpu.VMEM((1, _CH, 16), jnp.float32),
            pltpu.VMEM((1, 16, 16), jnp.float32),
            pltpu.VMEM((1, 16, 16), jnp.float32),
            pltpu.VMEM((_CH, 16), jnp.float32),
            pltpu.VMEM((_CH, 16), jnp.float32),
            pltpu.VMEM((1, _CH, 17), jnp.float32),
            pltpu.VMEM((1, _CH, 17), jnp.float32),
            pltpu.SemaphoreType.DMA,
            pltpu.SemaphoreType.DMA,
            pltpu.SemaphoreType.DMA,
            pltpu.SemaphoreType.DMA,
        ],
    )
    def k(raw_hbm, scr_hbm, anc_hbm, out_hbm, raw0, raw1, sc0, sc1, anc0,
          anc1, out0, out1, isem0, isem1, osem0, osem1):
        wid = lax.axis_index("s") * info.num_cores + lax.axis_index("c")
        bi = wid // halves
        a_base = (wid % halves) * per_w
        raw_v, sc_v, anc_v, out_v = (raw0, raw1), (sc0, sc1), (anc0, anc1), (out0, out1)
        isem, osem = (isem0, isem1), (osem0, osem1)

        lanes = jax.lax.iota(jnp.int32, 16)
        bit = lanes & 1
        lt4 = jnp.clip(4 - lanes, 0, 1)
        lt2 = jnp.clip(2 - lanes, 0, 1)
        mainperm = lt4 * (1 - bit) + (1 - lt4) * lanes
        partperm = lt4 * (3 - bit) + (1 - lt4) * lanes
        coef = 0.5 * (lt4 - 2 * lt2).astype(jnp.float32)
        even = 1 - bit
        ge4 = 1 - lt4
        xti = 1 - (even + ge4 - 2 * even * ge4)
        sidx = 3 - xti
        oidx = 1 - xti
        b2 = ((lanes >> 1) & 1) * lt4
        invmask = 1.0 - b2.astype(jnp.float32) * (1.0 - _INV)
        lt15 = jnp.clip(15 - lanes, 0, 1).astype(jnp.float32)
        shperm = (lanes + 1) & 15

        def _off(ci):
            return pl.multiple_of(a_base + ci * _CH, 8)

        def _sc_align(ci):
            off16 = (a_base + ci * _CH) // 16
            al = off16 - lax.rem(off16, 8)
            return pl.multiple_of(al, 8), off16 - al

        def in_copies(ci, bsel):
            off = _off(ci)
            al, _ = _sc_align(ci)
            return [
                pltpu.make_async_copy(
                    raw_hbm.at[pl.ds(bi, 1), pl.ds(off, _CH), :], raw_v[bsel],
                    isem[bsel]),
                pltpu.make_async_copy(
                    scr_hbm.at[pl.ds(bi, 1), pl.ds(al, 16), :],
                    sc_v[bsel], isem[bsel]),
                pltpu.make_async_copy(
                    anc_hbm.at[pl.ds(off, _CH), :], anc_v[bsel], isem[bsel]),
            ]

        def out_copy(ci, bsel):
            off = _off(ci)
            return pltpu.make_async_copy(
                out_v[bsel], out_hbm.at[pl.ds(bi, 1), pl.ds(off, _CH), :],
                osem[bsel])

        def start_in(ci, bsel):
            for c in in_copies(ci, bsel):
                c.start()

        def wait_in(ci, bsel):
            for c in in_copies(ci, bsel):
                c.wait()

        def compute(ci, bsel):
            rv, sv_ref, av, ov = raw_v[bsel], sc_v[bsel], anc_v[bsel], out_v[bsel]
            _, delta = _sc_align(ci)

            @plsc.parallel_loop(0, _CH, step=16)
            def _(g):
                sv = jnp.clip(
                    sv_ref[pl.ds(0, 1), pl.ds(delta + g // 16, 1), :].reshape(16),
                    -_CLIP, _CLIP)
                sig = 1.0 / (1.0 + jnp.exp(-sv))
                for r in range(16):
                    i = g + r
                    avec = av[pl.ds(i, 1), :].reshape(16) * invmask
                    v = rv[pl.ds(0, 1), pl.ds(i, 1), :].reshape(16)
                    main = jnp.take(v, mainperm)
                    part = jnp.take(v, partperm)
                    s = jnp.take(avec, sidx)
                    o = jnp.take(avec, oidx)
                    dec = s * (main + coef * part) + o
                    ov[pl.ds(0, 1), pl.ds(i, 1), 0:16] = dec.reshape(1, 1, 16)
                    sb = jnp.full((16,), sig[r], jnp.float32)
                    w = lt15 * jnp.take(dec, shperm) + (1.0 - lt15) * sb
                    ov[pl.ds(0, 1), pl.ds(i, 1), 1:17] = w.reshape(1, 1, 16)

        start_in(0, 0)
        even_end = n_chunks - (n_chunks % 2)

        @pl.loop(0, even_end, step=2)
        def _(ci0):
            for par in (0, 1):
                ci = ci0 + par

                @pl.when(ci + 1 < n_chunks)
                def _start_next():
                    start_in(ci + 1, 1 - par)

                wait_in(ci, bsel=par)

                @pl.when(ci >= 2)
                def _drain_out():
                    out_copy(ci - 2, par).wait()

                compute(ci, par)
                out_copy(ci, par).start()

        if n_chunks % 2:
            ci = n_chunks - 1
            par = ci % 2
            wait_in(ci, bsel=par)
            if n_chunks >= 3:
                out_copy(ci - 2, par).wait()
            compute(ci, par)
            out_copy(ci, par).start()

        out_copy(n_chunks - 2, (n_chunks - 2) % 2).wait()
        out_copy(n_chunks - 1, (n_chunks - 1) % 2).wait()

    return k(raw, scr, ancp)


@jax.jit
def _run(raw_box_tensor, raw_score_tensor, anchors):
    b, a, cin = raw_box_tensor.shape
    scr = jnp.pad(
        raw_score_tensor.reshape(b, a // 16, 16), ((0, 0), (0, 14), (0, 0)))
    ancp = jnp.pad(anchors, ((0, 0), (0, 12)))
    return _sc_run(raw_box_tensor, scr, ancp)


def kernel(raw_box_tensor, raw_score_tensor, anchors):
    return _run(raw_box_tensor, raw_score_tensor, anchors)
